# Initial kernel scaffold; baseline (speedup 1.0000x reference)
import jax
import jax.numpy as jnp
from jax.experimental import pallas as pl


def kernel(x, edge_index, W1, b1, Wmu, bmu, Wls, bls, eps):
    raise NotImplementedError("write your pallas kernel here")


# speedup vs baseline: 39.1924x; 39.1924x over previous
import functools

import jax
import jax.numpy as jnp
from jax import lax
from jax.experimental import pallas as pl
from jax.experimental.pallas import tpu as pltpu
from jax.experimental.pallas import tpu_sc as plsc

N = 10000
E = 320000
F_IN = 128
HID = 32
F_OUT = 16
NC, NS, L = 2, 16, 16
NW = NC * NS
CH = 128
K = -(-E // (NW * CH))
EPW = K * CH
N_ACC = 10240
RPT = N_ACC // NS
ZR = 40

_MESH = plsc.VectorSubcoreMesh(core_axis_name="c", subcore_axis_name="s")


@functools.partial(
    pl.kernel,
    out_type=jax.ShapeDtypeStruct((NC, N_ACC), jnp.float32),
    mesh=_MESH,
    scratch_types=[
        pltpu.VMEM((K, CH), jnp.int32),
        pltpu.VMEM((CH,), jnp.float32),
        pltpu.VMEM((RPT,), jnp.float32),
        pltpu.VMEM_SHARED((N_ACC,), jnp.float32),
    ],
)
def _deg_kernel(dstr, out, dst_v, ones_v, zbuf, acc):
    c = lax.axis_index("c")
    s = lax.axis_index("s")
    wid = s * NC + c

    def _fill(i, _):
        ones_v[pl.ds(i * L, L)] = jnp.full((L,), 1.0, jnp.float32)
        return 0

    lax.fori_loop(0, CH // L, _fill, 0)

    def _zb(i, _):
        zbuf[pl.ds(i * L, L)] = jnp.zeros((L,), jnp.float32)
        return 0

    lax.fori_loop(0, RPT // L, _zb, 0)
    pltpu.sync_copy(zbuf, acc.at[pl.ds(s * RPT, RPT)])
    pltpu.sync_copy(dstr.at[wid], dst_v)
    plsc.subcore_barrier()

    def _scat(k, _):
        pltpu.sync_copy(ones_v, acc.at[dst_v.at[k]], add=True)
        return 0

    lax.fori_loop(0, K, _scat, 0)
    plsc.subcore_barrier()
    pltpu.sync_copy(acc.at[pl.ds(s * RPT, RPT)], out.at[c, pl.ds(s * RPT, RPT)])


@functools.partial(
    pl.kernel,
    out_type=jax.ShapeDtypeStruct((NC, N_ACC, HID), jnp.float32),
    mesh=_MESH,
    scratch_types=[
        pltpu.VMEM((K, CH), jnp.int32),
        pltpu.VMEM((K, CH), jnp.int32),
        pltpu.VMEM((CH, HID), jnp.float32),
        pltpu.VMEM((ZR, HID), jnp.float32),
        pltpu.VMEM_SHARED((N_ACC, HID), jnp.float32),
    ],
    compiler_params=pltpu.CompilerParams(use_tc_tiling_on_sc=False),
)
def _spmm_kernel(table, srcr, dstr, out, src_v, dst_v, rowbuf, zbuf, acc):
    c = lax.axis_index("c")
    s = lax.axis_index("s")
    wid = s * NC + c

    def _zb(i, _):
        zbuf[i // 2, pl.ds((i % 2) * L, L)] = jnp.zeros((L,), jnp.float32)
        return 0

    lax.fori_loop(0, ZR * (HID // L), _zb, 0)

    def _zc(j, _):
        pltpu.sync_copy(zbuf, acc.at[pl.ds(s * RPT + j * ZR, ZR)])
        return 0

    lax.fori_loop(0, RPT // ZR, _zc, 0)
    pltpu.sync_copy(srcr.at[wid], src_v)
    pltpu.sync_copy(dstr.at[wid], dst_v)
    plsc.subcore_barrier()

    def _mb(k, _):
        pltpu.sync_copy(table.at[src_v.at[k]], rowbuf)
        pltpu.sync_copy(rowbuf, acc.at[dst_v.at[k]], add=True)
        return 0

    lax.fori_loop(0, K, _mb, 0)
    plsc.subcore_barrier()
    pltpu.sync_copy(acc.at[pl.ds(s * RPT, RPT)], out.at[c, pl.ds(s * RPT, RPT)])


BR = 512
GRID = N_ACC // BR


def _enc_body(degp_ref, x_ref, w1_ref, pt_ref, dinv_ref):
    deg = degp_ref[0, :] + degp_ref[1, :] + 1.0
    dinv = lax.rsqrt(jnp.maximum(deg, 1.0))
    p = jnp.dot(x_ref[...], w1_ref[...], preferred_element_type=jnp.float32)
    pt_ref[...] = p * dinv[:, None]
    dinv_ref[...] = dinv[:, None]


_enc = pl.pallas_call(
    _enc_body,
    grid=(GRID,),
    in_specs=[
        pl.BlockSpec((2, BR), lambda i: (0, i)),
        pl.BlockSpec((BR, F_IN), lambda i: (i, 0)),
        pl.BlockSpec((F_IN, HID), lambda i: (0, 0)),
    ],
    out_specs=[
        pl.BlockSpec((BR, HID), lambda i: (i, 0)),
        pl.BlockSpec((BR, 1), lambda i: (i, 0)),
    ],
    out_shape=[
        jax.ShapeDtypeStruct((N_ACC, HID), jnp.float32),
        jax.ShapeDtypeStruct((N_ACC, 1), jnp.float32),
    ],
)


def _mid_body(y_ref, pt_ref, dinv_ref, b1_ref, ht_ref):
    t = y_ref[0] + y_ref[1] + pt_ref[...]
    dinv = dinv_ref[...]
    h = jnp.maximum(dinv * t + b1_ref[...], 0.0)
    ht_ref[...] = h * dinv


_mid = pl.pallas_call(
    _mid_body,
    grid=(GRID,),
    in_specs=[
        pl.BlockSpec((2, BR, HID), lambda i: (0, i, 0)),
        pl.BlockSpec((BR, HID), lambda i: (i, 0)),
        pl.BlockSpec((BR, 1), lambda i: (i, 0)),
        pl.BlockSpec((1, HID), lambda i: (0, 0)),
    ],
    out_specs=pl.BlockSpec((BR, HID), lambda i: (i, 0)),
    out_shape=jax.ShapeDtypeStruct((N_ACC, HID), jnp.float32),
)


def _dec_body(z_ref, ht_ref, dinv_ref, wmu_ref, bmu_ref, wls_ref, bls_ref,
              eps_ref, zout_ref):
    agg = dinv_ref[...] * (z_ref[0] + z_ref[1] + ht_ref[...])
    mu = jnp.dot(agg, wmu_ref[...], preferred_element_type=jnp.float32) + bmu_ref[...]
    ls = jnp.dot(agg, wls_ref[...], preferred_element_type=jnp.float32) + bls_ref[...]
    zout_ref[...] = mu + eps_ref[...] * jnp.exp(ls)


_dec = pl.pallas_call(
    _dec_body,
    grid=(GRID,),
    in_specs=[
        pl.BlockSpec((2, BR, HID), lambda i: (0, i, 0)),
        pl.BlockSpec((BR, HID), lambda i: (i, 0)),
        pl.BlockSpec((BR, 1), lambda i: (i, 0)),
        pl.BlockSpec((HID, F_OUT), lambda i: (0, 0)),
        pl.BlockSpec((1, F_OUT), lambda i: (0, 0)),
        pl.BlockSpec((HID, F_OUT), lambda i: (0, 0)),
        pl.BlockSpec((1, F_OUT), lambda i: (0, 0)),
        pl.BlockSpec((BR, F_OUT), lambda i: (i, 0)),
    ],
    out_specs=pl.BlockSpec((BR, F_OUT), lambda i: (i, 0)),
    out_shape=jax.ShapeDtypeStruct((N, F_OUT), jnp.float32),
)


def kernel(x, edge_index, W1, b1, Wmu, bmu, Wls, bls, eps):
    src = edge_index[0]
    dst = edge_index[1]
    pad = NW * EPW - E
    ar = jnp.arange(pad, dtype=jnp.int32)
    pad_src = (ar * 37) % N
    pad_dst = N + ar % (N_ACC - N)
    srcr = jnp.concatenate([src, pad_src]).reshape(NW, K, CH)
    dstr = jnp.concatenate([dst, pad_dst]).reshape(NW, K, CH)

    degp = _deg_kernel(dstr)
    pt, dinv = _enc(degp, x, W1)
    y = _spmm_kernel(pt, srcr, dstr)
    ht = _mid(y, pt, dinv, b1.reshape(1, HID))
    z2 = _spmm_kernel(ht, srcr, dstr)
    return _dec(z2, ht, dinv, Wmu, bmu.reshape(1, F_OUT), Wls, bls.reshape(1, F_OUT), eps)


# speedup vs baseline: 53.0464x; 1.3535x over previous
import functools

import jax
import jax.numpy as jnp
from jax import lax
from jax.experimental import pallas as pl
from jax.experimental.pallas import tpu as pltpu
from jax.experimental.pallas import tpu_sc as plsc

N = 10000
E = 320000
F_IN = 128
HID = 32
F_OUT = 16
NC, NS, L = 2, 16, 16
NW = NC * NS
CH = 128
K = -(-E // (NW * CH))
EPW = K * CH
N_ACC = 10240
RPT = N_ACC // NS
ZR = 40

_MESH = plsc.VectorSubcoreMesh(core_axis_name="c", subcore_axis_name="s")


@functools.partial(
    pl.kernel,
    out_type=jax.ShapeDtypeStruct((NC, N_ACC), jnp.float32),
    mesh=_MESH,
    scratch_types=[
        pltpu.VMEM((K, CH), jnp.int32),
        pltpu.VMEM((CH,), jnp.float32),
        pltpu.VMEM((RPT,), jnp.float32),
        pltpu.VMEM_SHARED((N_ACC,), jnp.float32),
        pltpu.SemaphoreType.DMA,
    ],
)
def _deg_kernel(dstr, out, dst_v, ones_v, zbuf, acc, ssem):
    c = lax.axis_index("c")
    s = lax.axis_index("s")
    wid = s * NC + c

    def _fill(i, _):
        ones_v[pl.ds(i * L, L)] = jnp.full((L,), 1.0, jnp.float32)
        return 0

    lax.fori_loop(0, CH // L, _fill, 0)

    def _zb(i, _):
        zbuf[pl.ds(i * L, L)] = jnp.zeros((L,), jnp.float32)
        return 0

    lax.fori_loop(0, RPT // L, _zb, 0)
    pltpu.sync_copy(zbuf, acc.at[pl.ds(s * RPT, RPT)])
    pltpu.sync_copy(dstr.at[wid], dst_v)
    plsc.subcore_barrier()

    def _scat(k, _):
        @pl.when(k >= 4)
        def _():
            pltpu.make_async_copy(ones_v, acc.at[dst_v.at[k - 4]], ssem).wait()

        pltpu.async_copy(ones_v, acc.at[dst_v.at[k]], ssem, add=True)
        return 0

    lax.fori_loop(0, K, _scat, 0)

    def _drain(k, _):
        pltpu.make_async_copy(ones_v, acc.at[dst_v.at[k]], ssem).wait()
        return 0

    lax.fori_loop(K - 4, K, _drain, 0)
    plsc.subcore_barrier()
    pltpu.sync_copy(acc.at[pl.ds(s * RPT, RPT)], out.at[c, pl.ds(s * RPT, RPT)])


@functools.partial(
    pl.kernel,
    out_type=jax.ShapeDtypeStruct((NC, N_ACC, HID), jnp.float32),
    mesh=_MESH,
    scratch_types=[
        pltpu.VMEM((K, CH), jnp.int32),
        pltpu.VMEM((K, CH), jnp.int32),
        pltpu.VMEM((2, CH, HID), jnp.float32),
        pltpu.VMEM((ZR, HID), jnp.float32),
        pltpu.VMEM_SHARED((N_ACC, HID), jnp.float32),
        pltpu.SemaphoreType.DMA,
        pltpu.SemaphoreType.DMA,
    ],
    compiler_params=pltpu.CompilerParams(use_tc_tiling_on_sc=False),
)
def _spmm_kernel(table, srcr, dstr, out, src_v, dst_v, rowbuf, zbuf, acc, gsem, ssem):
    c = lax.axis_index("c")
    s = lax.axis_index("s")
    wid = s * NC + c

    def _zb(i, _):
        zbuf[i // 2, pl.ds((i % 2) * L, L)] = jnp.zeros((L,), jnp.float32)
        return 0

    lax.fori_loop(0, ZR * (HID // L), _zb, 0)

    def _zc(j, _):
        pltpu.sync_copy(zbuf, acc.at[pl.ds(s * RPT + j * ZR, ZR)])
        return 0

    lax.fori_loop(0, RPT // ZR, _zc, 0)
    pltpu.sync_copy(srcr.at[wid], src_v)
    pltpu.sync_copy(dstr.at[wid], dst_v)
    plsc.subcore_barrier()

    pltpu.async_copy(table.at[src_v.at[0]], rowbuf.at[0], gsem)

    def _mb(k, _):
        b = lax.rem(k, 2)
        nb = 1 - b

        @pl.when(k >= 1)
        def _():
            pltpu.make_async_copy(rowbuf.at[nb], acc.at[dst_v.at[k - 1]], ssem).wait()

        @pl.when(k + 1 < K)
        def _():
            pltpu.async_copy(table.at[src_v.at[k + 1]], rowbuf.at[nb], gsem)

        pltpu.make_async_copy(table.at[src_v.at[k]], rowbuf.at[b], gsem).wait()
        pltpu.async_copy(rowbuf.at[b], acc.at[dst_v.at[k]], ssem, add=True)
        return 0

    lax.fori_loop(0, K, _mb, 0)
    pltpu.make_async_copy(rowbuf.at[(K - 1) % 2], acc.at[dst_v.at[K - 1]], ssem).wait()
    plsc.subcore_barrier()
    pltpu.sync_copy(acc.at[pl.ds(s * RPT, RPT)], out.at[c, pl.ds(s * RPT, RPT)])


BR = 512
GRID = N_ACC // BR


def _enc_body(degp_ref, x_ref, w1_ref, pt_ref, dinv_ref):
    deg = degp_ref[0, :] + degp_ref[1, :] + 1.0
    dinv = lax.rsqrt(jnp.maximum(deg, 1.0))
    p = jnp.dot(x_ref[...], w1_ref[...], preferred_element_type=jnp.float32)
    pt_ref[...] = p * dinv[:, None]
    dinv_ref[...] = dinv[:, None]


_enc = pl.pallas_call(
    _enc_body,
    grid=(GRID,),
    in_specs=[
        pl.BlockSpec((2, BR), lambda i: (0, i)),
        pl.BlockSpec((BR, F_IN), lambda i: (i, 0)),
        pl.BlockSpec((F_IN, HID), lambda i: (0, 0)),
    ],
    out_specs=[
        pl.BlockSpec((BR, HID), lambda i: (i, 0)),
        pl.BlockSpec((BR, 1), lambda i: (i, 0)),
    ],
    out_shape=[
        jax.ShapeDtypeStruct((N_ACC, HID), jnp.float32),
        jax.ShapeDtypeStruct((N_ACC, 1), jnp.float32),
    ],
)


def _mid_body(y_ref, pt_ref, dinv_ref, b1_ref, ht_ref):
    t = y_ref[0] + y_ref[1] + pt_ref[...]
    dinv = dinv_ref[...]
    h = jnp.maximum(dinv * t + b1_ref[...], 0.0)
    ht_ref[...] = h * dinv


_mid = pl.pallas_call(
    _mid_body,
    grid=(GRID,),
    in_specs=[
        pl.BlockSpec((2, BR, HID), lambda i: (0, i, 0)),
        pl.BlockSpec((BR, HID), lambda i: (i, 0)),
        pl.BlockSpec((BR, 1), lambda i: (i, 0)),
        pl.BlockSpec((1, HID), lambda i: (0, 0)),
    ],
    out_specs=pl.BlockSpec((BR, HID), lambda i: (i, 0)),
    out_shape=jax.ShapeDtypeStruct((N_ACC, HID), jnp.float32),
)


def _dec_body(z_ref, ht_ref, dinv_ref, wmu_ref, bmu_ref, wls_ref, bls_ref,
              eps_ref, zout_ref):
    agg = dinv_ref[...] * (z_ref[0] + z_ref[1] + ht_ref[...])
    mu = jnp.dot(agg, wmu_ref[...], preferred_element_type=jnp.float32) + bmu_ref[...]
    ls = jnp.dot(agg, wls_ref[...], preferred_element_type=jnp.float32) + bls_ref[...]
    zout_ref[...] = mu + eps_ref[...] * jnp.exp(ls)


_dec = pl.pallas_call(
    _dec_body,
    grid=(GRID,),
    in_specs=[
        pl.BlockSpec((2, BR, HID), lambda i: (0, i, 0)),
        pl.BlockSpec((BR, HID), lambda i: (i, 0)),
        pl.BlockSpec((BR, 1), lambda i: (i, 0)),
        pl.BlockSpec((HID, F_OUT), lambda i: (0, 0)),
        pl.BlockSpec((1, F_OUT), lambda i: (0, 0)),
        pl.BlockSpec((HID, F_OUT), lambda i: (0, 0)),
        pl.BlockSpec((1, F_OUT), lambda i: (0, 0)),
        pl.BlockSpec((BR, F_OUT), lambda i: (i, 0)),
    ],
    out_specs=pl.BlockSpec((BR, F_OUT), lambda i: (i, 0)),
    out_shape=jax.ShapeDtypeStruct((N, F_OUT), jnp.float32),
)


def kernel(x, edge_index, W1, b1, Wmu, bmu, Wls, bls, eps):
    src = edge_index[0]
    dst = edge_index[1]
    pad = NW * EPW - E
    ar = jnp.arange(pad, dtype=jnp.int32)
    pad_src = (ar * 37) % N
    pad_dst = N + ar % (N_ACC - N)
    srcr = jnp.concatenate([src, pad_src]).reshape(NW, K, CH)
    dstr = jnp.concatenate([dst, pad_dst]).reshape(NW, K, CH)

    degp = _deg_kernel(dstr)
    pt, dinv = _enc(degp, x, W1)
    y = _spmm_kernel(pt, srcr, dstr)
    ht = _mid(y, pt, dinv, b1.reshape(1, HID))
    z2 = _spmm_kernel(ht, srcr, dstr)
    return _dec(z2, ht, dinv, Wmu, bmu.reshape(1, F_OUT), Wls, bls.reshape(1, F_OUT), eps)


# speedup vs baseline: 70.8718x; 1.3360x over previous
import functools

import jax
import jax.numpy as jnp
from jax import lax
from jax.experimental import pallas as pl
from jax.experimental.pallas import tpu as pltpu
from jax.experimental.pallas import tpu_sc as plsc

N = 10000
E = 320000
F_IN = 128
HID = 32
F_OUT = 16
NC, NS, L = 2, 16, 16
NW = NC * NS
CH = 128
K = -(-E // (NW * CH))
EPW = K * CH
N_ACC = 10240
RPT = N_ACC // NS
ZR = 40
NB = 4
GRID = 8
BR = N_ACC // GRID

_MESH = plsc.VectorSubcoreMesh(core_axis_name="c", subcore_axis_name="s")


@functools.partial(
    pl.kernel,
    out_type=jax.ShapeDtypeStruct((NC, N_ACC), jnp.float32),
    mesh=_MESH,
    scratch_types=[
        pltpu.VMEM((K, CH), jnp.int32),
        pltpu.VMEM((CH,), jnp.float32),
        pltpu.VMEM((RPT,), jnp.float32),
        pltpu.VMEM_SHARED((N_ACC,), jnp.float32),
        pltpu.SemaphoreType.DMA,
    ],
)
def _deg_kernel(dstr, out, dst_v, ones_v, zbuf, acc, ssem):
    c = lax.axis_index("c")
    s = lax.axis_index("s")
    wid = s * NC + c

    def _fill(i, _):
        ones_v[pl.ds(i * L, L)] = jnp.full((L,), 1.0, jnp.float32)
        return 0

    lax.fori_loop(0, CH // L, _fill, 0)

    def _zb(i, _):
        zbuf[pl.ds(i * L, L)] = jnp.zeros((L,), jnp.float32)
        return 0

    lax.fori_loop(0, RPT // L, _zb, 0)
    pltpu.sync_copy(zbuf, acc.at[pl.ds(s * RPT, RPT)])
    pltpu.sync_copy(dstr.at[wid], dst_v)
    plsc.subcore_barrier()

    def _scat(k, _):
        @pl.when(k >= 4)
        def _():
            pltpu.make_async_copy(ones_v, acc.at[dst_v.at[k - 4]], ssem).wait()

        pltpu.async_copy(ones_v, acc.at[dst_v.at[k]], ssem, add=True)
        return 0

    lax.fori_loop(0, K, _scat, 0)

    def _drain(k, _):
        pltpu.make_async_copy(ones_v, acc.at[dst_v.at[k]], ssem).wait()
        return 0

    lax.fori_loop(K - 4, K, _drain, 0)
    plsc.subcore_barrier()
    pltpu.sync_copy(acc.at[pl.ds(s * RPT, RPT)], out.at[c, pl.ds(s * RPT, RPT)])


@functools.partial(
    pl.kernel,
    out_type=jax.ShapeDtypeStruct((NC, N_ACC, HID), jnp.float32),
    mesh=_MESH,
    scratch_types=[
        pltpu.VMEM((K, CH), jnp.int32),
        pltpu.VMEM((K, CH), jnp.int32),
        pltpu.VMEM((NB, CH, HID), jnp.float32),
        pltpu.VMEM((ZR, HID), jnp.float32),
        pltpu.VMEM_SHARED((N_ACC, HID), jnp.float32),
        pltpu.SemaphoreType.DMA,
        pltpu.SemaphoreType.DMA,
    ],
    compiler_params=pltpu.CompilerParams(use_tc_tiling_on_sc=False),
)
def _spmm_kernel(table, srcr, dstr, out, src_v, dst_v, rowbuf, zbuf, acc, gsem, ssem):
    c = lax.axis_index("c")
    s = lax.axis_index("s")
    wid = s * NC + c

    def _zb(i, _):
        zbuf[i // 2, pl.ds((i % 2) * L, L)] = jnp.zeros((L,), jnp.float32)
        return 0

    lax.fori_loop(0, ZR * (HID // L), _zb, 0)

    def _zc(j, _):
        pltpu.sync_copy(zbuf, acc.at[pl.ds(s * RPT + j * ZR, ZR)])
        return 0

    lax.fori_loop(0, RPT // ZR, _zc, 0)
    pltpu.sync_copy(srcr.at[wid], src_v)
    pltpu.sync_copy(dstr.at[wid], dst_v)
    plsc.subcore_barrier()

    for j in range(NB - 1):
        pltpu.async_copy(table.at[src_v.at[j]], rowbuf.at[j], gsem)

    def _mb(k, _):
        b = lax.rem(k, NB)
        pb = lax.rem(k + NB - 1, NB)

        @pl.when(k >= 1)
        def _():
            pltpu.make_async_copy(rowbuf.at[pb], acc.at[dst_v.at[k - 1]], ssem).wait()

        @pl.when(k + NB - 1 < K)
        def _():
            pltpu.async_copy(table.at[src_v.at[k + NB - 1]], rowbuf.at[pb], gsem)

        pltpu.make_async_copy(table.at[src_v.at[k]], rowbuf.at[b], gsem).wait()
        pltpu.async_copy(rowbuf.at[b], acc.at[dst_v.at[k]], ssem, add=True)
        return 0

    lax.fori_loop(0, K, _mb, 0)
    pltpu.make_async_copy(rowbuf.at[(K - 1) % NB], acc.at[dst_v.at[K - 1]], ssem).wait()
    plsc.subcore_barrier()
    pltpu.sync_copy(acc.at[pl.ds(s * RPT, RPT)], out.at[c, pl.ds(s * RPT, RPT)])


def _enc_body(degp_ref, x_ref, w1_ref, pt_ref, dinv_ref):
    deg = degp_ref[0, :] + degp_ref[1, :] + 1.0
    dinv = lax.rsqrt(jnp.maximum(deg, 1.0))[:, None]
    p = jnp.dot(x_ref[...], w1_ref[...], preferred_element_type=jnp.float32)
    pt_ref[pl.ds(0, N)] = p * dinv[:N]
    pt_ref[pl.ds(N, N_ACC - N)] = jnp.zeros((N_ACC - N, HID), jnp.float32)
    dinv_ref[...] = dinv


_enc = pl.pallas_call(
    _enc_body,
    out_shape=[
        jax.ShapeDtypeStruct((N_ACC, HID), jnp.float32),
        jax.ShapeDtypeStruct((N_ACC, 1), jnp.float32),
    ],
)


def _mid_body(y_ref, pt_ref, dinv_ref, b1_ref, ht_ref):
    t = y_ref[0] + y_ref[1] + pt_ref[...]
    dinv = dinv_ref[...]
    h = jnp.maximum(dinv * t + b1_ref[...], 0.0)
    ht_ref[...] = h * dinv


_mid = pl.pallas_call(
    _mid_body,
    grid=(GRID,),
    in_specs=[
        pl.BlockSpec((2, BR, HID), lambda i: (0, i, 0)),
        pl.BlockSpec((BR, HID), lambda i: (i, 0)),
        pl.BlockSpec((BR, 1), lambda i: (i, 0)),
        pl.BlockSpec((1, HID), lambda i: (0, 0)),
    ],
    out_specs=pl.BlockSpec((BR, HID), lambda i: (i, 0)),
    out_shape=jax.ShapeDtypeStruct((N_ACC, HID), jnp.float32),
)


def _dec_body(z_ref, ht_ref, dinv_ref, wmu_ref, bmu_ref, wls_ref, bls_ref,
              eps_ref, zout_ref):
    agg = dinv_ref[...] * (z_ref[0] + z_ref[1] + ht_ref[...])
    mu = jnp.dot(agg, wmu_ref[...], preferred_element_type=jnp.float32) + bmu_ref[...]
    ls = jnp.dot(agg, wls_ref[...], preferred_element_type=jnp.float32) + bls_ref[...]
    zout_ref[...] = mu + eps_ref[...] * jnp.exp(ls)


_dec = pl.pallas_call(
    _dec_body,
    grid=(GRID,),
    in_specs=[
        pl.BlockSpec((2, BR, HID), lambda i: (0, i, 0)),
        pl.BlockSpec((BR, HID), lambda i: (i, 0)),
        pl.BlockSpec((BR, 1), lambda i: (i, 0)),
        pl.BlockSpec((HID, F_OUT), lambda i: (0, 0)),
        pl.BlockSpec((1, F_OUT), lambda i: (0, 0)),
        pl.BlockSpec((HID, F_OUT), lambda i: (0, 0)),
        pl.BlockSpec((1, F_OUT), lambda i: (0, 0)),
        pl.BlockSpec((BR, F_OUT), lambda i: (i, 0)),
    ],
    out_specs=pl.BlockSpec((BR, F_OUT), lambda i: (i, 0)),
    out_shape=jax.ShapeDtypeStruct((N, F_OUT), jnp.float32),
)


def kernel(x, edge_index, W1, b1, Wmu, bmu, Wls, bls, eps):
    src = edge_index[0]
    dst = edge_index[1]
    pad = NW * EPW - E
    ar = jnp.arange(pad, dtype=jnp.int32)
    pad_src = (ar * 37) % N
    pad_dst = N + ar % (N_ACC - N)
    srcr = jnp.concatenate([src, pad_src]).reshape(NW, K, CH)
    dstr = jnp.concatenate([dst, pad_dst]).reshape(NW, K, CH)

    degp = _deg_kernel(dstr)
    pt, dinv = _enc(degp, x, W1)
    y = _spmm_kernel(pt, srcr, dstr)
    ht = _mid(y, pt, dinv, b1.reshape(1, HID))
    z2 = _spmm_kernel(ht, srcr, dstr)
    return _dec(z2, ht, dinv, Wmu, bmu.reshape(1, F_OUT), Wls, bls.reshape(1, F_OUT), eps)


# speedup vs baseline: 73.3184x; 1.0345x over previous
import functools

import jax
import jax.numpy as jnp
from jax import lax
from jax.experimental import pallas as pl
from jax.experimental.pallas import tpu as pltpu
from jax.experimental.pallas import tpu_sc as plsc

N = 10000
E = 320000
F_IN = 128
HID = 32
F_OUT = 16
NC, NS, L = 2, 16, 16
NW = NC * NS
CH = 128
K = -(-E // (NW * CH))
EPW = K * CH
N_ACC = 10240
RPT = N_ACC // NS
ZR = 40
NB = 6
GRID = 8
BR = N_ACC // GRID

_MESH = plsc.VectorSubcoreMesh(core_axis_name="c", subcore_axis_name="s")


@functools.partial(
    pl.kernel,
    out_type=jax.ShapeDtypeStruct((NC, N_ACC), jnp.float32),
    mesh=_MESH,
    scratch_types=[
        pltpu.VMEM((K, CH), jnp.int32),
        pltpu.VMEM((CH,), jnp.float32),
        pltpu.VMEM((RPT,), jnp.float32),
        pltpu.VMEM_SHARED((N_ACC,), jnp.float32),
        pltpu.SemaphoreType.DMA,
    ],
)
def _deg_kernel(dstr, out, dst_v, ones_v, zbuf, acc, ssem):
    c = lax.axis_index("c")
    s = lax.axis_index("s")
    wid = s * NC + c

    def _fill(i, _):
        ones_v[pl.ds(i * L, L)] = jnp.full((L,), 1.0, jnp.float32)
        return 0

    lax.fori_loop(0, CH // L, _fill, 0)

    def _zb(i, _):
        zbuf[pl.ds(i * L, L)] = jnp.zeros((L,), jnp.float32)
        return 0

    lax.fori_loop(0, RPT // L, _zb, 0)
    pltpu.sync_copy(zbuf, acc.at[pl.ds(s * RPT, RPT)])
    pltpu.sync_copy(dstr.at[wid], dst_v)
    plsc.subcore_barrier()

    def _scat(k, _):
        @pl.when(k >= 4)
        def _():
            pltpu.make_async_copy(ones_v, acc.at[dst_v.at[k - 4]], ssem).wait()

        pltpu.async_copy(ones_v, acc.at[dst_v.at[k]], ssem, add=True)
        return 0

    lax.fori_loop(0, K, _scat, 0)

    def _drain(k, _):
        pltpu.make_async_copy(ones_v, acc.at[dst_v.at[k]], ssem).wait()
        return 0

    lax.fori_loop(K - 4, K, _drain, 0)
    plsc.subcore_barrier()
    pltpu.sync_copy(acc.at[pl.ds(s * RPT, RPT)], out.at[c, pl.ds(s * RPT, RPT)])


@functools.partial(
    pl.kernel,
    out_type=jax.ShapeDtypeStruct((NC, N_ACC, HID), jnp.float32),
    mesh=_MESH,
    scratch_types=[
        pltpu.VMEM((K, CH), jnp.int32),
        pltpu.VMEM((K, CH), jnp.int32),
        pltpu.VMEM((NB, CH, HID), jnp.float32),
        pltpu.VMEM((ZR, HID), jnp.float32),
        pltpu.VMEM_SHARED((N_ACC, HID), jnp.float32),
        pltpu.SemaphoreType.DMA,
        pltpu.SemaphoreType.DMA,
    ],
    compiler_params=pltpu.CompilerParams(use_tc_tiling_on_sc=False),
)
def _spmm_kernel(table, srcr, dstr, out, src_v, dst_v, rowbuf, zbuf, acc, gsem, ssem):
    c = lax.axis_index("c")
    s = lax.axis_index("s")
    wid = s * NC + c

    def _zb(i, _):
        zbuf[i // 2, pl.ds((i % 2) * L, L)] = jnp.zeros((L,), jnp.float32)
        return 0

    lax.fori_loop(0, ZR * (HID // L), _zb, 0)

    def _zc(j, _):
        pltpu.sync_copy(zbuf, acc.at[pl.ds(s * RPT + j * ZR, ZR)])
        return 0

    lax.fori_loop(0, RPT // ZR, _zc, 0)
    pltpu.sync_copy(srcr.at[wid], src_v)
    pltpu.sync_copy(dstr.at[wid], dst_v)
    plsc.subcore_barrier()

    for j in range(NB - 1):
        pltpu.async_copy(table.at[src_v.at[j]], rowbuf.at[j], gsem)

    def _mb(k, _):
        b = lax.rem(k, NB)
        pb = lax.rem(k + NB - 1, NB)

        @pl.when(k >= 1)
        def _():
            pltpu.make_async_copy(rowbuf.at[pb], acc.at[dst_v.at[k - 1]], ssem).wait()

        @pl.when(k + NB - 1 < K)
        def _():
            pltpu.async_copy(table.at[src_v.at[k + NB - 1]], rowbuf.at[pb], gsem)

        pltpu.make_async_copy(table.at[src_v.at[k]], rowbuf.at[b], gsem).wait()
        pltpu.async_copy(rowbuf.at[b], acc.at[dst_v.at[k]], ssem, add=True)
        return 0

    lax.fori_loop(0, K, _mb, 0)
    pltpu.make_async_copy(rowbuf.at[(K - 1) % NB], acc.at[dst_v.at[K - 1]], ssem).wait()
    plsc.subcore_barrier()
    pltpu.sync_copy(acc.at[pl.ds(s * RPT, RPT)], out.at[c, pl.ds(s * RPT, RPT)])


def _enc_body(degp_ref, x_ref, w1_ref, pt_ref, dinv_ref):
    deg = degp_ref[0, :] + degp_ref[1, :] + 1.0
    dinv = lax.rsqrt(jnp.maximum(deg, 1.0))[:, None]
    p = jnp.dot(x_ref[...], w1_ref[...], preferred_element_type=jnp.float32)
    pt_ref[pl.ds(0, N)] = p * dinv[:N]
    pt_ref[pl.ds(N, N_ACC - N)] = jnp.zeros((N_ACC - N, HID), jnp.float32)
    dinv_ref[...] = dinv


_enc = pl.pallas_call(
    _enc_body,
    out_shape=[
        jax.ShapeDtypeStruct((N_ACC, HID), jnp.float32),
        jax.ShapeDtypeStruct((N_ACC, 1), jnp.float32),
    ],
)


def _mid_body(y_ref, pt_ref, dinv_ref, b1_ref, ht_ref):
    t = y_ref[0] + y_ref[1] + pt_ref[...]
    dinv = dinv_ref[...]
    h = jnp.maximum(dinv * t + b1_ref[...], 0.0)
    ht_ref[...] = h * dinv


_mid = pl.pallas_call(
    _mid_body,
    grid=(GRID,),
    in_specs=[
        pl.BlockSpec((2, BR, HID), lambda i: (0, i, 0)),
        pl.BlockSpec((BR, HID), lambda i: (i, 0)),
        pl.BlockSpec((BR, 1), lambda i: (i, 0)),
        pl.BlockSpec((1, HID), lambda i: (0, 0)),
    ],
    out_specs=pl.BlockSpec((BR, HID), lambda i: (i, 0)),
    out_shape=jax.ShapeDtypeStruct((N_ACC, HID), jnp.float32),
)


def _dec_body(z_ref, ht_ref, dinv_ref, wmu_ref, bmu_ref, wls_ref, bls_ref,
              eps_ref, zout_ref):
    agg = dinv_ref[...] * (z_ref[0] + z_ref[1] + ht_ref[...])
    mu = jnp.dot(agg, wmu_ref[...], preferred_element_type=jnp.float32) + bmu_ref[...]
    ls = jnp.dot(agg, wls_ref[...], preferred_element_type=jnp.float32) + bls_ref[...]
    zout_ref[...] = mu + eps_ref[...] * jnp.exp(ls)


_dec = pl.pallas_call(
    _dec_body,
    grid=(GRID,),
    in_specs=[
        pl.BlockSpec((2, BR, HID), lambda i: (0, i, 0)),
        pl.BlockSpec((BR, HID), lambda i: (i, 0)),
        pl.BlockSpec((BR, 1), lambda i: (i, 0)),
        pl.BlockSpec((HID, F_OUT), lambda i: (0, 0)),
        pl.BlockSpec((1, F_OUT), lambda i: (0, 0)),
        pl.BlockSpec((HID, F_OUT), lambda i: (0, 0)),
        pl.BlockSpec((1, F_OUT), lambda i: (0, 0)),
        pl.BlockSpec((BR, F_OUT), lambda i: (i, 0)),
    ],
    out_specs=pl.BlockSpec((BR, F_OUT), lambda i: (i, 0)),
    out_shape=jax.ShapeDtypeStruct((N, F_OUT), jnp.float32),
)


def kernel(x, edge_index, W1, b1, Wmu, bmu, Wls, bls, eps):
    src = edge_index[0]
    dst = edge_index[1]
    pad = NW * EPW - E
    ar = jnp.arange(pad, dtype=jnp.int32)
    pad_src = (ar * 37) % N
    pad_dst = N + ar % (N_ACC - N)
    srcr = jnp.concatenate([src, pad_src]).reshape(NW, K, CH)
    dstr = jnp.concatenate([dst, pad_dst]).reshape(NW, K, CH)

    degp = _deg_kernel(dstr)
    pt, dinv = _enc(degp, x, W1)
    y = _spmm_kernel(pt, srcr, dstr)
    ht = _mid(y, pt, dinv, b1.reshape(1, HID))
    z2 = _spmm_kernel(ht, srcr, dstr)
    return _dec(z2, ht, dinv, Wmu, bmu.reshape(1, F_OUT), Wls, bls.reshape(1, F_OUT), eps)


# speedup vs baseline: 73.4720x; 1.0021x over previous
import functools

import jax
import jax.numpy as jnp
from jax import lax
from jax.experimental import pallas as pl
from jax.experimental.pallas import tpu as pltpu
from jax.experimental.pallas import tpu_sc as plsc

N = 10000
E = 320000
F_IN = 128
HID = 32
F_OUT = 16
NC, NS, L = 2, 16, 16
NW = NC * NS
CH = 128
K = -(-E // (NW * CH))
EPW = K * CH
N_ACC = 10240
RPT = N_ACC // NS
ZR = 40
NB = 6
GRID = 8
BR = N_ACC // GRID

_MESH = plsc.VectorSubcoreMesh(core_axis_name="c", subcore_axis_name="s")


@functools.partial(
    pl.kernel,
    out_type=jax.ShapeDtypeStruct((NC, N_ACC), jnp.float32),
    mesh=_MESH,
    scratch_types=[
        pltpu.VMEM((K, CH), jnp.int32),
        pltpu.VMEM((CH,), jnp.float32),
        pltpu.VMEM((RPT,), jnp.float32),
        pltpu.VMEM_SHARED((N_ACC,), jnp.float32),
        pltpu.SemaphoreType.DMA,
    ],
)
def _deg_kernel(dstr, out, dst_v, ones_v, zbuf, acc, ssem):
    c = lax.axis_index("c")
    s = lax.axis_index("s")
    wid = s * NC + c

    def _fill(i, _):
        ones_v[pl.ds(i * L, L)] = jnp.full((L,), 1.0, jnp.float32)
        return 0

    lax.fori_loop(0, CH // L, _fill, 0)

    def _zb(i, _):
        zbuf[pl.ds(i * L, L)] = jnp.zeros((L,), jnp.float32)
        return 0

    lax.fori_loop(0, RPT // L, _zb, 0)
    pltpu.sync_copy(zbuf, acc.at[pl.ds(s * RPT, RPT)])
    pltpu.sync_copy(dstr.at[wid], dst_v)
    plsc.subcore_barrier()

    def _scat(k, _):
        @pl.when(k >= 4)
        def _():
            pltpu.make_async_copy(ones_v, acc.at[dst_v.at[k - 4]], ssem).wait()

        pltpu.async_copy(ones_v, acc.at[dst_v.at[k]], ssem, add=True)
        return 0

    lax.fori_loop(0, K, _scat, 0)

    def _drain(k, _):
        pltpu.make_async_copy(ones_v, acc.at[dst_v.at[k]], ssem).wait()
        return 0

    lax.fori_loop(K - 4, K, _drain, 0)
    plsc.subcore_barrier()
    pltpu.sync_copy(acc.at[pl.ds(s * RPT, RPT)], out.at[c, pl.ds(s * RPT, RPT)])


@functools.partial(
    pl.kernel,
    out_type=jax.ShapeDtypeStruct((NC, N_ACC, HID), jnp.float32),
    mesh=_MESH,
    scratch_types=[
        pltpu.VMEM((K, CH), jnp.int32),
        pltpu.VMEM((K, CH), jnp.int32),
        pltpu.VMEM((NB, CH, HID), jnp.float32),
        pltpu.VMEM((ZR, HID), jnp.float32),
        pltpu.VMEM_SHARED((N_ACC, HID), jnp.float32),
        pltpu.SemaphoreType.DMA,
        pltpu.SemaphoreType.DMA,
    ],
    compiler_params=pltpu.CompilerParams(use_tc_tiling_on_sc=False),
)
def _spmm_kernel(table, srcr, dstr, out, src_v, dst_v, rowbuf, zbuf, acc, gsem, ssem):
    c = lax.axis_index("c")
    s = lax.axis_index("s")
    wid = s * NC + c

    def _zb(i, _):
        zbuf[i // 2, pl.ds((i % 2) * L, L)] = jnp.zeros((L,), jnp.float32)
        return 0

    lax.fori_loop(0, ZR * (HID // L), _zb, 0)

    def _zc(j, _):
        pltpu.sync_copy(zbuf, acc.at[pl.ds(s * RPT + j * ZR, ZR)])
        return 0

    lax.fori_loop(0, RPT // ZR, _zc, 0)
    pltpu.sync_copy(srcr.at[wid], src_v)
    pltpu.sync_copy(dstr.at[wid], dst_v)
    plsc.subcore_barrier()

    for j in range(NB - 2):
        pltpu.async_copy(table.at[src_v.at[j]], rowbuf.at[j], gsem)

    def _mb(k, _):
        b = lax.rem(k, NB)
        pb = lax.rem(k + NB - 2, NB)

        @pl.when(k >= 2)
        def _():
            pltpu.make_async_copy(rowbuf.at[pb], acc.at[dst_v.at[k - 2]], ssem).wait()

        @pl.when(k + NB - 2 < K)
        def _():
            pltpu.async_copy(table.at[src_v.at[k + NB - 2]], rowbuf.at[pb], gsem)

        pltpu.make_async_copy(table.at[src_v.at[k]], rowbuf.at[b], gsem).wait()
        pltpu.async_copy(rowbuf.at[b], acc.at[dst_v.at[k]], ssem, add=True)
        return 0

    lax.fori_loop(0, K, _mb, 0)
    pltpu.make_async_copy(rowbuf.at[(K - 2) % NB], acc.at[dst_v.at[K - 2]], ssem).wait()
    pltpu.make_async_copy(rowbuf.at[(K - 1) % NB], acc.at[dst_v.at[K - 1]], ssem).wait()
    plsc.subcore_barrier()
    pltpu.sync_copy(acc.at[pl.ds(s * RPT, RPT)], out.at[c, pl.ds(s * RPT, RPT)])


def _enc_body(degp_ref, x_ref, w1_ref, pt_ref, dinv_ref):
    deg = degp_ref[0, :] + degp_ref[1, :] + 1.0
    dinv = lax.rsqrt(jnp.maximum(deg, 1.0))[:, None]
    p = jnp.dot(x_ref[...], w1_ref[...], preferred_element_type=jnp.float32)
    pt_ref[pl.ds(0, N)] = p * dinv[:N]
    pt_ref[pl.ds(N, N_ACC - N)] = jnp.zeros((N_ACC - N, HID), jnp.float32)
    dinv_ref[...] = dinv


_enc = pl.pallas_call(
    _enc_body,
    out_shape=[
        jax.ShapeDtypeStruct((N_ACC, HID), jnp.float32),
        jax.ShapeDtypeStruct((N_ACC, 1), jnp.float32),
    ],
)


def _mid_body(y_ref, pt_ref, dinv_ref, b1_ref, ht_ref):
    t = y_ref[0] + y_ref[1] + pt_ref[...]
    dinv = dinv_ref[...]
    h = jnp.maximum(dinv * t + b1_ref[...], 0.0)
    ht_ref[...] = h * dinv


_mid = pl.pallas_call(
    _mid_body,
    grid=(GRID,),
    in_specs=[
        pl.BlockSpec((2, BR, HID), lambda i: (0, i, 0)),
        pl.BlockSpec((BR, HID), lambda i: (i, 0)),
        pl.BlockSpec((BR, 1), lambda i: (i, 0)),
        pl.BlockSpec((1, HID), lambda i: (0, 0)),
    ],
    out_specs=pl.BlockSpec((BR, HID), lambda i: (i, 0)),
    out_shape=jax.ShapeDtypeStruct((N_ACC, HID), jnp.float32),
)


def _dec_body(z_ref, ht_ref, dinv_ref, wmu_ref, bmu_ref, wls_ref, bls_ref,
              eps_ref, zout_ref):
    agg = dinv_ref[...] * (z_ref[0] + z_ref[1] + ht_ref[...])
    mu = jnp.dot(agg, wmu_ref[...], preferred_element_type=jnp.float32) + bmu_ref[...]
    ls = jnp.dot(agg, wls_ref[...], preferred_element_type=jnp.float32) + bls_ref[...]
    zout_ref[...] = mu + eps_ref[...] * jnp.exp(ls)


_dec = pl.pallas_call(
    _dec_body,
    grid=(GRID,),
    in_specs=[
        pl.BlockSpec((2, BR, HID), lambda i: (0, i, 0)),
        pl.BlockSpec((BR, HID), lambda i: (i, 0)),
        pl.BlockSpec((BR, 1), lambda i: (i, 0)),
        pl.BlockSpec((HID, F_OUT), lambda i: (0, 0)),
        pl.BlockSpec((1, F_OUT), lambda i: (0, 0)),
        pl.BlockSpec((HID, F_OUT), lambda i: (0, 0)),
        pl.BlockSpec((1, F_OUT), lambda i: (0, 0)),
        pl.BlockSpec((BR, F_OUT), lambda i: (i, 0)),
    ],
    out_specs=pl.BlockSpec((BR, F_OUT), lambda i: (i, 0)),
    out_shape=jax.ShapeDtypeStruct((N, F_OUT), jnp.float32),
)


def kernel(x, edge_index, W1, b1, Wmu, bmu, Wls, bls, eps):
    src = edge_index[0]
    dst = edge_index[1]
    pad = NW * EPW - E
    ar = jnp.arange(pad, dtype=jnp.int32)
    pad_src = (ar * 37) % N
    pad_dst = N + ar % (N_ACC - N)
    srcr = jnp.concatenate([src, pad_src]).reshape(NW, K, CH)
    dstr = jnp.concatenate([dst, pad_dst]).reshape(NW, K, CH)

    degp = _deg_kernel(dstr)
    pt, dinv = _enc(degp, x, W1)
    y = _spmm_kernel(pt, srcr, dstr)
    ht = _mid(y, pt, dinv, b1.reshape(1, HID))
    z2 = _spmm_kernel(ht, srcr, dstr)
    return _dec(z2, ht, dinv, Wmu, bmu.reshape(1, F_OUT), Wls, bls.reshape(1, F_OUT), eps)
